# baseline (speedup 1.0000x reference)
import functools

import jax
import jax.numpy as jnp
from jax import lax
from jax.experimental import pallas as pl
from jax.experimental.pallas import tpu as pltpu
from jax.experimental.pallas import tpu_sc as plsc

N_PTS = 10000
N_NB = 32
K_PTS = 15
D_IN = 128
D_OUT = 128
NTOT = N_PTS * N_NB

NW = 32
CH = 128
PER_W = NTOT // NW
N_CHUNK = (PER_W + CH - 1) // CH - 1

BLOCK = 1000
GRP = 8


def _sc_gather_body(xtab, sxt, syt, szt, nbf, nf_out, sx_out, sy_out, sz_out,
                    idx_v, rows_a, rows_b, cox_v, coy_v, coz_v,
                    sem_a, sem_b, sem_c):
    wid = lax.axis_index("s") * 2 + lax.axis_index("c")
    base = wid * PER_W
    pltpu.sync_copy(nbf.at[pl.ds(base, PER_W)], idx_v)

    def chunk_off(c):
        return pl.multiple_of(jnp.minimum(c * CH, PER_W - CH), 8)

    def start(c, buf, sem):
        off = chunk_off(c)
        return pltpu.async_copy(xtab.at[idx_v.at[pl.ds(off, CH)]], buf, sem)

    def finish(c, buf, sem):
        off = chunk_off(c)
        idxs = idx_v.at[pl.ds(off, CH)]
        cx = pltpu.async_copy(sxt.at[idxs], cox_v.at[pl.ds(off, CH)], sem_c)
        cy = pltpu.async_copy(syt.at[idxs], coy_v.at[pl.ds(off, CH)], sem_c)
        cz = pltpu.async_copy(szt.at[idxs], coz_v.at[pl.ds(off, CH)], sem_c)
        pltpu.make_async_copy(xtab.at[idx_v.at[pl.ds(0, CH)]], buf, sem).wait()
        pltpu.sync_copy(buf, nf_out.at[pl.ds(base + off, CH)])
        cx.wait()
        cy.wait()
        cz.wait()

    start(0, rows_a, sem_a)

    def pair(i, carry):
        start(2 * i + 1, rows_b, sem_b)
        finish(2 * i, rows_a, sem_a)
        start(jnp.minimum(2 * i + 2, N_CHUNK), rows_a, sem_a)
        finish(2 * i + 1, rows_b, sem_b)
        return carry

    lax.fori_loop(0, (N_CHUNK + 1) // 2, pair, 0)
    finish(N_CHUNK, rows_a, sem_a)
    pltpu.sync_copy(cox_v, sx_out.at[pl.ds(base, PER_W)])
    pltpu.sync_copy(coy_v, sy_out.at[pl.ds(base, PER_W)])
    pltpu.sync_copy(coz_v, sz_out.at[pl.ds(base, PER_W)])


def _sc_gather(xtab, sxt, syt, szt, nbf):
    mesh = plsc.VectorSubcoreMesh(core_axis_name="c", subcore_axis_name="s")
    fn = pl.kernel(
        _sc_gather_body,
        mesh=mesh,
        out_type=[
            jax.ShapeDtypeStruct((NTOT, D_IN), jnp.float32),
            jax.ShapeDtypeStruct((NTOT,), jnp.float32),
            jax.ShapeDtypeStruct((NTOT,), jnp.float32),
            jax.ShapeDtypeStruct((NTOT,), jnp.float32),
        ],
        scratch_types=[
            pltpu.VMEM((PER_W,), jnp.int32),
            pltpu.VMEM((CH, D_IN), jnp.float32),
            pltpu.VMEM((CH, D_IN), jnp.float32),
            pltpu.VMEM((PER_W,), jnp.float32),
            pltpu.VMEM((PER_W,), jnp.float32),
            pltpu.VMEM((PER_W,), jnp.float32),
            pltpu.SemaphoreType.DMA,
            pltpu.SemaphoreType.DMA,
            pltpu.SemaphoreType.DMA,
        ],
    )
    return fn(xtab, sxt, syt, szt, nbf)


def _tc_body(nf_ref, sgt_ref, qrep_ref, kt_ref, mask_ref,
             w_ref, out_ref, wf_ref):
    rows = BLOCK * N_NB
    qr = qrep_ref[...]
    sg = sgt_ref[...]
    ux = sg[0:1, :] - qr[0:1, :]
    uy = sg[1:2, :] - qr[1:2, :]
    uz = sg[2:3, :] - qr[2:3, :]
    rho = ux * ux + uy * uy + uz * uz
    kx = kt_ref[:, 0:1]
    ky = kt_ref[:, 1:2]
    kz = kt_ref[:, 2:3]
    k2 = kt_ref[:, 3:4]
    d2 = (rho + k2) - 2.0 * (kx * ux + ky * uy + kz * uz)
    d2 = jnp.maximum(d2, 0.0)
    w = jnp.maximum(1.0 - jnp.sqrt(d2), 0.0)

    nfb = nf_ref[...].astype(jnp.bfloat16)
    mask = mask_ref[...]

    n_grp = BLOCK // GRP
    span = GRP * N_NB
    for g in range(n_grp):
        wg = w[:, g * span : (g + 1) * span]
        wrep = jnp.broadcast_to(wg[:, None, :], (16, GRP, span))
        wrep = wrep.reshape(16 * GRP, span)
        wmask = (wrep * mask).astype(jnp.bfloat16)
        nfg = nfb[g * span : (g + 1) * span, :]
        m = jnp.dot(wmask, nfg, preferred_element_type=jnp.float32)
        for k in range(K_PTS):
            wf_ref[k, g * GRP : (g + 1) * GRP, :] = m[k * GRP : (k + 1) * GRP, :]

    acc = jnp.zeros((BLOCK, D_OUT), dtype=jnp.float32)
    for k in range(K_PTS):
        acc = acc + jnp.dot(
            wf_ref[k], w_ref[k], preferred_element_type=jnp.float32
        )
    out_ref[...] = acc


def _tc_compute(nf, sgt, qrep, kt, mask, weight, interpret=False):
    n = nf.shape[0] // N_NB
    grid = (n // BLOCK,)
    rows = BLOCK * N_NB
    return pl.pallas_call(
        _tc_body,
        grid=grid,
        in_specs=[
            pl.BlockSpec((rows, D_IN), lambda i: (i, 0)),
            pl.BlockSpec((3, rows), lambda i: (0, i)),
            pl.BlockSpec((3, rows), lambda i: (0, i)),
            pl.BlockSpec((16, 8), lambda i: (0, 0)),
            pl.BlockSpec((16 * GRP, GRP * N_NB), lambda i: (0, 0)),
            pl.BlockSpec((K_PTS, D_IN, D_OUT), lambda i: (0, 0, 0)),
        ],
        out_specs=pl.BlockSpec((BLOCK, D_OUT), lambda i: (i, 0)),
        out_shape=jax.ShapeDtypeStruct((n, D_OUT), jnp.float32),
        scratch_shapes=[pltpu.VMEM((K_PTS, BLOCK, D_OUT), jnp.float32)],
        interpret=interpret,
    )(nf, sgt, qrep, kt, mask, weight)


def _prep(query_points, support_points, x, K_points):
    xtab = jnp.concatenate(
        [x, jnp.zeros((8, D_IN), jnp.float32)], axis=0
    )
    ctail = jnp.concatenate(
        [jnp.full((1,), 1e6, jnp.float32), jnp.zeros((7,), jnp.float32)]
    )
    sxt = jnp.concatenate([support_points[:, 0], ctail])
    syt = jnp.concatenate([support_points[:, 1], ctail])
    szt = jnp.concatenate([support_points[:, 2], ctail])
    qrep = jnp.repeat(jnp.transpose(query_points), N_NB, axis=1)
    k2 = jnp.sum(K_points * K_points, axis=1, keepdims=True)
    kt = jnp.concatenate([K_points, k2], axis=1)
    kt = jnp.concatenate([kt, jnp.full((1, 4), 1e6, jnp.float32)], axis=0)
    kt = jnp.concatenate([kt, jnp.zeros((16, 4), jnp.float32)], axis=1)
    r = jax.lax.broadcasted_iota(jnp.int32, (16 * GRP, GRP * N_NB), 0)
    c = jax.lax.broadcasted_iota(jnp.int32, (16 * GRP, GRP * N_NB), 1)
    mask = ((r % GRP) == (c // N_NB)).astype(jnp.float32)
    return xtab, sxt, syt, szt, qrep, kt, mask


@jax.jit
def kernel(query_points, support_points, neighbors, x, K_points, weight):
    xtab, sxt, syt, szt, qrep, kt, mask = _prep(
        query_points, support_points, x, K_points
    )
    nbf = neighbors.reshape(-1)
    nf, sx_g, sy_g, sz_g = _sc_gather(xtab, sxt, syt, szt, nbf)
    sgt = jnp.stack([sx_g, sy_g, sz_g])
    return _tc_compute(nf, sgt, qrep, kt, mask, weight)


# speedup vs baseline: 1.0620x; 1.0620x over previous
import functools

import jax
import jax.numpy as jnp
from jax import lax
from jax.experimental import pallas as pl
from jax.experimental.pallas import tpu as pltpu
from jax.experimental.pallas import tpu_sc as plsc

N_PTS = 10000
N_NB = 32
K_PTS = 15
D_IN = 128
D_OUT = 128
NTOT = N_PTS * N_NB

NW = 32
CH = 128
PER_W = NTOT // NW
N_CHUNK = (PER_W + CH - 1) // CH - 1

BLOCK = 400
GRP = 8


def _sc_gather_body(xtab, sxt, syt, szt, nbf, nf_out, sx_out, sy_out, sz_out,
                    idx_v, rows_a, rows_b, cox_v, coy_v, coz_v,
                    sem_a, sem_b, sem_c):
    wid = lax.axis_index("s") * 2 + lax.axis_index("c")
    base = wid * PER_W
    pltpu.sync_copy(nbf.at[pl.ds(base, PER_W)], idx_v)

    def chunk_off(c):
        return pl.multiple_of(jnp.minimum(c * CH, PER_W - CH), 8)

    def start(c, buf, sem):
        off = chunk_off(c)
        return pltpu.async_copy(xtab.at[idx_v.at[pl.ds(off, CH)]], buf, sem)

    def finish(c, buf, sem):
        off = chunk_off(c)
        idxs = idx_v.at[pl.ds(off, CH)]
        cx = pltpu.async_copy(sxt.at[idxs], cox_v.at[pl.ds(off, CH)], sem_c)
        cy = pltpu.async_copy(syt.at[idxs], coy_v.at[pl.ds(off, CH)], sem_c)
        cz = pltpu.async_copy(szt.at[idxs], coz_v.at[pl.ds(off, CH)], sem_c)
        pltpu.make_async_copy(xtab.at[idx_v.at[pl.ds(0, CH)]], buf, sem).wait()
        pltpu.sync_copy(buf, nf_out.at[pl.ds(base + off, CH)])
        cx.wait()
        cy.wait()
        cz.wait()

    start(0, rows_a, sem_a)

    def pair(i, carry):
        start(2 * i + 1, rows_b, sem_b)
        finish(2 * i, rows_a, sem_a)
        start(jnp.minimum(2 * i + 2, N_CHUNK), rows_a, sem_a)
        finish(2 * i + 1, rows_b, sem_b)
        return carry

    lax.fori_loop(0, (N_CHUNK + 1) // 2, pair, 0)
    finish(N_CHUNK, rows_a, sem_a)
    pltpu.sync_copy(cox_v, sx_out.at[pl.ds(base, PER_W)])
    pltpu.sync_copy(coy_v, sy_out.at[pl.ds(base, PER_W)])
    pltpu.sync_copy(coz_v, sz_out.at[pl.ds(base, PER_W)])


def _sc_gather(xtab, sxt, syt, szt, nbf):
    mesh = plsc.VectorSubcoreMesh(core_axis_name="c", subcore_axis_name="s")
    fn = pl.kernel(
        _sc_gather_body,
        mesh=mesh,
        out_type=[
            jax.ShapeDtypeStruct((NTOT, D_IN), jnp.float32),
            jax.ShapeDtypeStruct((NTOT,), jnp.float32),
            jax.ShapeDtypeStruct((NTOT,), jnp.float32),
            jax.ShapeDtypeStruct((NTOT,), jnp.float32),
        ],
        scratch_types=[
            pltpu.VMEM((PER_W,), jnp.int32),
            pltpu.VMEM((CH, D_IN), jnp.float32),
            pltpu.VMEM((CH, D_IN), jnp.float32),
            pltpu.VMEM((PER_W,), jnp.float32),
            pltpu.VMEM((PER_W,), jnp.float32),
            pltpu.VMEM((PER_W,), jnp.float32),
            pltpu.SemaphoreType.DMA,
            pltpu.SemaphoreType.DMA,
            pltpu.SemaphoreType.DMA,
        ],
    )
    return fn(xtab, sxt, syt, szt, nbf)


def _tc_body(nf_ref, sgt_ref, qt_ref, kt_ref, mask_ref,
             w_ref, out_ref, wf_ref):
    rows = BLOCK * N_NB
    qt = qt_ref[0]
    qr = jnp.broadcast_to(qt[:, :, None], (3, BLOCK, N_NB)).reshape(3, rows)
    sg = sgt_ref[...]
    ux = sg[0:1, :] - qr[0:1, :]
    uy = sg[1:2, :] - qr[1:2, :]
    uz = sg[2:3, :] - qr[2:3, :]
    rho = ux * ux + uy * uy + uz * uz
    kx = kt_ref[:, 0:1]
    ky = kt_ref[:, 1:2]
    kz = kt_ref[:, 2:3]
    k2 = kt_ref[:, 3:4]
    d2 = (rho + k2) - 2.0 * (kx * ux + ky * uy + kz * uz)
    d2 = jnp.maximum(d2, 0.0)
    w = jnp.maximum(1.0 - jnp.sqrt(d2), 0.0)

    nfb = nf_ref[...].astype(jnp.bfloat16)
    mask = mask_ref[...]

    n_grp = BLOCK // GRP
    span = GRP * N_NB
    for g in range(n_grp):
        wg = w[:, g * span : (g + 1) * span]
        wrep = jnp.broadcast_to(wg[:, None, :], (16, GRP, span))
        wrep = wrep.reshape(16 * GRP, span)
        wmask = (wrep * mask).astype(jnp.bfloat16)
        nfg = nfb[g * span : (g + 1) * span, :]
        m = jnp.dot(wmask, nfg, preferred_element_type=jnp.float32)
        for k in range(K_PTS):
            wf_ref[k, g * GRP : (g + 1) * GRP, :] = m[k * GRP : (k + 1) * GRP, :]

    acc = jnp.zeros((BLOCK, D_OUT), dtype=jnp.float32)
    for k in range(K_PTS):
        acc = acc + jnp.dot(
            wf_ref[k], w_ref[k], preferred_element_type=jnp.float32
        )
    out_ref[...] = acc


def _tc_compute(nf, sgt, qt, kt, mask, weight, interpret=False):
    n = nf.shape[0] // N_NB
    grid = (n // BLOCK,)
    rows = BLOCK * N_NB
    return pl.pallas_call(
        _tc_body,
        grid=grid,
        in_specs=[
            pl.BlockSpec((rows, D_IN), lambda i: (i, 0)),
            pl.BlockSpec((3, rows), lambda i: (0, i)),
            pl.BlockSpec((1, 3, BLOCK), lambda i: (i, 0, 0)),
            pl.BlockSpec((16, 8), lambda i: (0, 0)),
            pl.BlockSpec((16 * GRP, GRP * N_NB), lambda i: (0, 0)),
            pl.BlockSpec((K_PTS, D_IN, D_OUT), lambda i: (0, 0, 0)),
        ],
        out_specs=pl.BlockSpec((BLOCK, D_OUT), lambda i: (i, 0)),
        out_shape=jax.ShapeDtypeStruct((n, D_OUT), jnp.float32),
        scratch_shapes=[pltpu.VMEM((K_PTS, BLOCK, D_OUT), jnp.float32)],
        interpret=interpret,
    )(nf, sgt, qt, kt, mask, weight)


def _prep(query_points, support_points, x, K_points):
    xtab = jnp.concatenate(
        [x, jnp.zeros((8, D_IN), jnp.float32)], axis=0
    )
    ctail = jnp.concatenate(
        [jnp.full((1,), 1e6, jnp.float32), jnp.zeros((7,), jnp.float32)]
    )
    sxt = jnp.concatenate([support_points[:, 0], ctail])
    syt = jnp.concatenate([support_points[:, 1], ctail])
    szt = jnp.concatenate([support_points[:, 2], ctail])
    qt = jnp.transpose(
        jnp.transpose(query_points).reshape(3, N_PTS // BLOCK, BLOCK),
        (1, 0, 2),
    )
    k2 = jnp.sum(K_points * K_points, axis=1, keepdims=True)
    kt = jnp.concatenate([K_points, k2], axis=1)
    kt = jnp.concatenate([kt, jnp.full((1, 4), 1e6, jnp.float32)], axis=0)
    kt = jnp.concatenate([kt, jnp.zeros((16, 4), jnp.float32)], axis=1)
    r = jax.lax.broadcasted_iota(jnp.int32, (16 * GRP, GRP * N_NB), 0)
    c = jax.lax.broadcasted_iota(jnp.int32, (16 * GRP, GRP * N_NB), 1)
    mask = ((r % GRP) == (c // N_NB)).astype(jnp.float32)
    return xtab, sxt, syt, szt, qt, kt, mask


@jax.jit
def kernel(query_points, support_points, neighbors, x, K_points, weight):
    xtab, sxt, syt, szt, qt, kt, mask = _prep(
        query_points, support_points, x, K_points
    )
    nbf = neighbors.reshape(-1)
    nf, sx_g, sy_g, sz_g = _sc_gather(xtab, sxt, syt, szt, nbf)
    sgt = jnp.stack([sx_g, sy_g, sz_g])
    return _tc_compute(nf, sgt, qt, kt, mask, weight)
